# Initial kernel scaffold; baseline (speedup 1.0000x reference)
import jax
import jax.numpy as jnp
from jax.experimental import pallas as pl


def kernel(tgt, table):
    raise NotImplementedError("write your pallas kernel here")


# speedup vs baseline: 2.0139x; 2.0139x over previous
import functools
import math

import numpy as np
import jax
import jax.numpy as jnp
from jax import lax
from jax.experimental import pallas as pl
from jax.experimental.pallas import tpu as pltpu
from jax.experimental.pallas import tpu_sc as plsc

D_MODEL = 256
BATCH = 1024
SEQ = 50
B = SEQ * BATCH
NC, NS = 2, 16
NW = NC * NS
ROWS_PER_W = B // NW
CHUNK = 64
NCHUNKS = ROWS_PER_W // CHUNK
LANES = 16
NVEC = D_MODEL // LANES
SCALE = 16.0


def _pe_rows():
    position = np.arange(SEQ, dtype=np.float32)[:, None]
    div_term = np.exp(
        np.arange(0, D_MODEL, 2, dtype=np.float32) * -(math.log(10000.0) / D_MODEL)
    )
    pe = np.zeros((SEQ, D_MODEL), dtype=np.float32)
    pe[:, 0::2] = np.sin(position * div_term)
    pe[:, 1::2] = np.cos(position * div_term)
    return pe


_PE = _pe_rows()


def _sc_embed(table, idx_flat, pe):
    mesh = plsc.VectorSubcoreMesh(core_axis_name="c", subcore_axis_name="s")

    @functools.partial(
        pl.kernel,
        mesh=mesh,
        out_type=jax.ShapeDtypeStruct((B, D_MODEL), jnp.float32),
        scratch_types=[
            pltpu.VMEM((ROWS_PER_W,), jnp.int32),
            pltpu.VMEM((CHUNK, D_MODEL), jnp.float32),
            pltpu.VMEM((SEQ, D_MODEL), jnp.float32),
            pltpu.SemaphoreType.DMA,
        ],
    )
    def k(table_hbm, idx_hbm, pe_hbm, out_hbm, idx_v, rows_v, pe_v, sem):
        wid = lax.axis_index("s") * NC + lax.axis_index("c")
        base = pl.multiple_of(wid * ROWS_PER_W, ROWS_PER_W)
        pltpu.sync_copy(pe_hbm, pe_v)
        pltpu.sync_copy(idx_hbm.at[pl.ds(base, ROWS_PER_W)], idx_v)

        def chunk_body(ci, _):
            loff = pl.multiple_of(ci * CHUNK, CHUNK)
            goff = pl.multiple_of(base + loff, CHUNK)
            t = goff // BATCH
            pltpu.async_copy(
                table_hbm.at[idx_v.at[pl.ds(loff, CHUNK)]], rows_v, sem
            ).wait()
            pe_vecs = [pe_v[t, pl.ds(j * LANES, LANES)] for j in range(NVEC)]

            def row_body(r, _):
                for j in range(NVEC):
                    sl = pl.ds(j * LANES, LANES)
                    rows_v[r, sl] = rows_v[r, sl] * SCALE + pe_vecs[j]
                return 0

            lax.fori_loop(0, CHUNK, row_body, 0)
            pltpu.sync_copy(rows_v, out_hbm.at[pl.ds(goff, CHUNK)])
            return 0

        lax.fori_loop(0, NCHUNKS, chunk_body, 0)

    return k(table, idx_flat, pe)


def _masks(tgt32):
    def body(tgt_ref, pad_ref, tri_ref):
        pad_ref[...] = tgt_ref[...] == 0
        r = lax.broadcasted_iota(jnp.int32, (SEQ, SEQ), 0)
        c = lax.broadcasted_iota(jnp.int32, (SEQ, SEQ), 1)
        tri_ref[...] = jnp.where(c <= r, 0.0, -jnp.inf).astype(jnp.float32)

    return pl.pallas_call(
        body,
        out_shape=(
            jax.ShapeDtypeStruct((BATCH, SEQ), jnp.bool_),
            jax.ShapeDtypeStruct((SEQ, SEQ), jnp.float32),
        ),
    )(tgt32)


def kernel(tgt, table):
    tgt32 = tgt.astype(jnp.int32)
    idx_flat = jnp.transpose(tgt32).reshape(B)
    emb_flat = _sc_embed(table, idx_flat, jnp.asarray(_PE))
    pad, tri = _masks(tgt32)
    return emb_flat.reshape(SEQ, BATCH, D_MODEL), tri, pad


# speedup vs baseline: 2.8347x; 1.4076x over previous
import functools
import math

import numpy as np
import jax
import jax.numpy as jnp
from jax import lax
from jax.experimental import pallas as pl
from jax.experimental.pallas import tpu as pltpu
from jax.experimental.pallas import tpu_sc as plsc

D_MODEL = 256
BATCH = 1024
SEQ = 50
B = SEQ * BATCH
NC, NS = 2, 16
NW = NC * NS
ROWS_PER_W = B // NW
CHUNK = 64
NCHUNKS = ROWS_PER_W // CHUNK
LANES = 16
NVEC = D_MODEL // LANES
SCALE = 16.0


def _pe_rows():
    position = np.arange(SEQ, dtype=np.float32)[:, None]
    div_term = np.exp(
        np.arange(0, D_MODEL, 2, dtype=np.float32) * -(math.log(10000.0) / D_MODEL)
    )
    pe = np.zeros((SEQ, D_MODEL), dtype=np.float32)
    pe[:, 0::2] = np.sin(position * div_term)
    pe[:, 1::2] = np.cos(position * div_term)
    return pe


_PE = _pe_rows()


NBUF = 3


def _sc_embed(table, idx_flat, pe):
    mesh = plsc.VectorSubcoreMesh(core_axis_name="c", subcore_axis_name="s")

    @functools.partial(
        pl.kernel,
        mesh=mesh,
        out_type=jax.ShapeDtypeStruct((B, D_MODEL), jnp.float32),
        scratch_types=[
            pltpu.VMEM((ROWS_PER_W,), jnp.int32),
            *[pltpu.VMEM((CHUNK, D_MODEL), jnp.float32) for _ in range(NBUF)],
            pltpu.VMEM((SEQ, D_MODEL), jnp.float32),
            *[pltpu.SemaphoreType.DMA for _ in range(2 * NBUF)],
        ],
    )
    def k(table_hbm, idx_hbm, pe_hbm, out_hbm, idx_v, *rest):
        bufs = list(rest[:NBUF])
        pe_v = rest[NBUF]
        gsems = list(rest[NBUF + 1 : NBUF + 1 + NBUF])
        ssems = list(rest[NBUF + 1 + NBUF :])

        wid = lax.axis_index("s") * NC + lax.axis_index("c")
        base = pl.multiple_of(wid * ROWS_PER_W, ROWS_PER_W)
        pltpu.sync_copy(pe_hbm, pe_v)
        pltpu.sync_copy(idx_hbm.at[pl.ds(base, ROWS_PER_W)], idx_v)

        def start_gather(ci):
            bi = ci % NBUF
            return pltpu.async_copy(
                table_hbm.at[idx_v.at[pl.ds(ci * CHUNK, CHUNK)]], bufs[bi], gsems[bi]
            )

        def compute(ci):
            buf = bufs[ci % NBUF]
            t = (base + ci * CHUNK) // BATCH
            pe_vecs = [pe_v[t, pl.ds(j * LANES, LANES)] for j in range(NVEC)]

            def row_body(r, _):
                for j in range(NVEC):
                    sl = pl.ds(j * LANES, LANES)
                    buf[r, sl] = buf[r, sl] * SCALE + pe_vecs[j]
                return 0

            lax.fori_loop(0, CHUNK, row_body, 0)

        gcp, scp, waited = {}, {}, set()
        for ci in range(min(2, NCHUNKS)):
            gcp[ci] = start_gather(ci)
        for ci in range(NCHUNKS):
            bi = ci % NBUF
            nxt = ci + 2
            if nxt < NCHUNKS:
                prev = nxt - NBUF
                if prev >= 0:
                    scp[prev].wait()
                    waited.add(prev)
                gcp[nxt] = start_gather(nxt)
            gcp[ci].wait()
            compute(ci)
            scp[ci] = pltpu.async_copy(
                bufs[bi],
                out_hbm.at[pl.ds(pl.multiple_of(base + ci * CHUNK, CHUNK), CHUNK)],
                ssems[bi],
            )
        for ci in range(NCHUNKS):
            if ci not in waited:
                scp[ci].wait()

    return k(table, idx_flat, pe)


def _masks(tgt32):
    def body(tgt_ref, pad_ref, tri_ref):
        pad_ref[...] = tgt_ref[...] == 0
        r = lax.broadcasted_iota(jnp.int32, (SEQ, SEQ), 0)
        c = lax.broadcasted_iota(jnp.int32, (SEQ, SEQ), 1)
        tri_ref[...] = jnp.where(c <= r, 0.0, -jnp.inf).astype(jnp.float32)

    return pl.pallas_call(
        body,
        out_shape=(
            jax.ShapeDtypeStruct((BATCH, SEQ), jnp.bool_),
            jax.ShapeDtypeStruct((SEQ, SEQ), jnp.float32),
        ),
    )(tgt32)


def kernel(tgt, table):
    tgt32 = tgt.astype(jnp.int32)
    idx_flat = jnp.transpose(tgt32).reshape(B)
    emb_flat = _sc_embed(table, idx_flat, jnp.asarray(_PE))
    pad, tri = _masks(tgt32)
    return emb_flat.reshape(SEQ, BATCH, D_MODEL), tri, pad
